# baseline (speedup 1.0000x reference)
import functools

import jax
import jax.numpy as jnp
from jax import lax
from jax.experimental import pallas as pl
from jax.experimental.pallas import tpu as pltpu
from jax.experimental.pallas import tpu_sc as plsc

_ROWS = 2048
_NC, _NS, _L = 2, 16, 16
_NW = _NC * _NS


def _probs_kernel(x_ref, w_ref, probs_ref, probst_ref):
    x = x_ref[...]
    w = w_ref[...]
    logits = jax.lax.dot_general(
        x, w, (((1,), (1,)), ((), ())),
        preferred_element_type=jnp.float32,
        precision=jax.lax.Precision.DEFAULT,
    )
    m = jnp.max(logits, axis=-1, keepdims=True)
    e = jnp.exp(logits - m)
    s = jnp.sum(e, axis=-1, keepdims=True)
    probs = e / s
    probs_ref[...] = probs
    probst_ref[...] = probs.T


def _make_sc_top2(n_tokens, n_experts):
    tpw = n_tokens // _NW
    mesh = plsc.VectorSubcoreMesh(core_axis_name="c", subcore_axis_name="s")

    @functools.partial(
        pl.kernel, mesh=mesh,
        out_type=[
            jax.ShapeDtypeStruct((2, n_tokens), jnp.int32),
            jax.ShapeDtypeStruct((2, n_tokens), jnp.float32),
        ],
        scratch_types=[
            pltpu.VMEM((n_experts, tpw), jnp.float32),
            pltpu.VMEM((2, tpw), jnp.int32),
            pltpu.VMEM((2, tpw), jnp.float32),
        ],
    )
    def sc_top2(probst_hbm, idx_hbm, wts_hbm, probs_v, idx_v, wts_v):
        wid = lax.axis_index("s") * _NC + lax.axis_index("c")
        base = wid * tpw
        pltpu.sync_copy(probst_hbm.at[:, pl.ds(base, tpw)], probs_v)
        emask = n_experts - 1

        def group_body(g, carry):
            sl = pl.ds(g * _L, _L)
            v1 = jnp.full((_L,), -1.0, jnp.float32)
            v2 = jnp.full((_L,), -1.0, jnp.float32)
            i1 = jnp.zeros((_L,), jnp.int32)
            i2 = jnp.zeros((_L,), jnp.int32)
            for e in range(n_experts):
                r = probs_v[e, sl]
                ev = jnp.full((_L,), e, jnp.int32)
                gt1 = r > v1
                nv2 = jnp.where(gt1, v1, r)
                ni2 = jnp.where(gt1, i1, ev)
                upd2 = nv2 > v2
                v2 = jnp.where(upd2, nv2, v2)
                i2 = jnp.where(upd2, ni2, i2)
                v1 = jnp.where(gt1, r, v1)
                i1 = jnp.where(gt1, ev, i1)
            idx_v[0, sl] = i1
            idx_v[1, sl] = i2
            den = v1 + v2 + jnp.float32(1e-9)
            wts_v[0, sl] = v1 / den
            wts_v[1, sl] = v2 / den
            return carry

        lax.fori_loop(0, tpw // _L, group_body, 0)

        pltpu.sync_copy(idx_v.at[0], idx_hbm.at[0, pl.ds(base, tpw)])
        pltpu.sync_copy(idx_v.at[1], idx_hbm.at[1, pl.ds(base, tpw)])
        pltpu.sync_copy(wts_v.at[0], wts_hbm.at[0, pl.ds(base, tpw)])
        pltpu.sync_copy(wts_v.at[1], wts_hbm.at[1, pl.ds(base, tpw)])

    return sc_top2


def kernel(x, W):
    B, T, D = x.shape
    N = B * T
    E = W.shape[0]
    x2 = x.reshape(N, D)
    R = _ROWS
    probs, probs_t = pl.pallas_call(
        _probs_kernel,
        grid=(N // R,),
        in_specs=[
            pl.BlockSpec((R, D), lambda i: (i, 0)),
            pl.BlockSpec((E, D), lambda i: (0, 0)),
        ],
        out_specs=[
            pl.BlockSpec((R, E), lambda i: (i, 0)),
            pl.BlockSpec((E, R), lambda i: (0, i)),
        ],
        out_shape=[
            jax.ShapeDtypeStruct((N, E), jnp.float32),
            jax.ShapeDtypeStruct((E, N), jnp.float32),
        ],
    )(x2, W)
    idx_t, wts_t = _make_sc_top2(N, E)(probs_t)
    return (probs, idx_t.T, wts_t.T)


# speedup vs baseline: 1.2788x; 1.2788x over previous
import jax
import jax.numpy as jnp
from jax.experimental import pallas as pl

_ROWS = 2048


def _router_kernel(x_ref, w_ref, probs_ref, small_ref):
    x = x_ref[...]
    w = w_ref[...]
    logits = jax.lax.dot_general(
        x, w, (((1,), (1,)), ((), ())),
        preferred_element_type=jnp.float32,
        precision=jax.lax.Precision.DEFAULT,
    )
    m = jnp.max(logits, axis=-1, keepdims=True)
    e = jnp.exp(logits - m)
    s = jnp.sum(e, axis=-1, keepdims=True)
    probs = e / s
    probs_ref[...] = probs

    ncols = probs.shape[-1]
    iota = jax.lax.broadcasted_iota(jnp.int32, probs.shape, 1)
    key = (jax.lax.bitcast_convert_type(probs, jnp.int32) | (ncols - 1)) - iota
    k1 = jnp.max(key, axis=-1, keepdims=True)
    k2 = jnp.max(jnp.where(key == k1, 0, key), axis=-1, keepdims=True)
    idx1 = (ncols - 1) - (k1 & (ncols - 1))
    idx2 = (ncols - 1) - (k2 & (ncols - 1))
    p1 = jax.lax.bitcast_convert_type(k1 | (ncols - 1), jnp.float32)
    p2 = jax.lax.bitcast_convert_type(k2 | (ncols - 1), jnp.float32)
    denom = p1 + p2 + jnp.float32(1e-9)
    small = jnp.concatenate(
        [idx1.astype(jnp.float32), idx2.astype(jnp.float32),
         p1 / denom, p2 / denom], axis=-1)
    small_ref[0, :, :] = small.T


def kernel(x, W):
    B, T, D = x.shape
    N = B * T
    E = W.shape[0]
    x2 = x.reshape(N, D)
    R = _ROWS
    nblk = N // R
    probs, small = pl.pallas_call(
        _router_kernel,
        grid=(nblk,),
        in_specs=[
            pl.BlockSpec((R, D), lambda i: (i, 0)),
            pl.BlockSpec((E, D), lambda i: (0, 0)),
        ],
        out_specs=[
            pl.BlockSpec((R, E), lambda i: (i, 0)),
            pl.BlockSpec((1, 4, R), lambda i: (i, 0, 0)),
        ],
        out_shape=[
            jax.ShapeDtypeStruct((N, E), jnp.float32),
            jax.ShapeDtypeStruct((nblk, 4, R), jnp.float32),
        ],
    )(x2, W)
    sm = jnp.transpose(small, (0, 2, 1)).reshape(N, 4)
    idx = sm[:, 0:2].astype(jnp.int32)
    wts = sm[:, 2:4]
    return (probs, idx, wts)


# speedup vs baseline: 1.2846x; 1.0045x over previous
import jax
import jax.numpy as jnp
from jax.experimental import pallas as pl
from jax.experimental.pallas import tpu as pltpu

_ROWS = 2048


def _router_kernel(x_ref, w_ref, probs_ref, small_ref):
    x = x_ref[...]
    w = w_ref[...]
    logits = jax.lax.dot_general(
        x, w, (((1,), (1,)), ((), ())),
        preferred_element_type=jnp.float32,
        precision=jax.lax.Precision.DEFAULT,
    )
    m = jnp.max(logits, axis=-1, keepdims=True)
    e = jnp.exp(logits - m)
    s = jnp.sum(e, axis=-1, keepdims=True)
    probs = e / s
    probs_ref[...] = probs

    ncols = probs.shape[-1]
    iota = jax.lax.broadcasted_iota(jnp.int32, probs.shape, 1)
    key = (jax.lax.bitcast_convert_type(probs, jnp.int32) | (ncols - 1)) - iota
    k1 = jnp.max(key, axis=-1, keepdims=True)
    k2 = jnp.max(jnp.where(key == k1, 0, key), axis=-1, keepdims=True)
    idx1 = (ncols - 1) - (k1 & (ncols - 1))
    idx2 = (ncols - 1) - (k2 & (ncols - 1))
    p1 = jax.lax.bitcast_convert_type(k1 | (ncols - 1), jnp.float32)
    p2 = jax.lax.bitcast_convert_type(k2 | (ncols - 1), jnp.float32)
    denom = p1 + p2 + jnp.float32(1e-9)
    small = jnp.concatenate(
        [idx1.astype(jnp.float32), idx2.astype(jnp.float32),
         p1 / denom, p2 / denom], axis=-1)
    small_ref[0, :, :] = small.T


def kernel(x, W):
    B, T, D = x.shape
    N = B * T
    E = W.shape[0]
    x2 = x.reshape(N, D)
    R = _ROWS
    nblk = N // R
    probs, small = pl.pallas_call(
        _router_kernel,
        grid=(nblk,),
        in_specs=[
            pl.BlockSpec((R, D), lambda i: (i, 0)),
            pl.BlockSpec((E, D), lambda i: (0, 0)),
        ],
        out_specs=[
            pl.BlockSpec((R, E), lambda i: (i, 0)),
            pl.BlockSpec((1, 4, R), lambda i: (i, 0, 0)),
        ],
        out_shape=[
            jax.ShapeDtypeStruct((N, E), jnp.float32),
            jax.ShapeDtypeStruct((nblk, 4, R), jnp.float32),
        ],
        compiler_params=pltpu.CompilerParams(
            dimension_semantics=("parallel",)),
    )(x2, W)
    sm = jnp.transpose(small, (0, 2, 1)).reshape(N, 4)
    idx = sm[:, 0:2].astype(jnp.int32)
    wts = sm[:, 2:4]
    return (probs, idx, wts)
